# Initial kernel scaffold; baseline (speedup 1.0000x reference)
import jax
import jax.numpy as jnp
from jax.experimental import pallas as pl


def kernel(h, e, edge_index, snorm_n, snorm_e, params):
    raise NotImplementedError("write your pallas kernel here")


# speedup vs baseline: 2.5847x; 2.5847x over previous
import functools

import jax
import jax.numpy as jnp
import numpy as np
from jax import lax
from jax.experimental import pallas as pl
from jax.experimental.pallas import tpu as pltpu
from jax.experimental.pallas import tpu_sc as plsc

_N = 10000
_E = 160000
_H = 128
_L = 4
_AVG_D_LOG = float(np.log(17.0))

_NC, _NS = 2, 16
_NW = _NC * _NS
_NPW = 320
_NPAD = _NW * _NPW
_CHUNK = 160
_EB = 128
_BN = 256
_NEG = -3.0e38
_POS = 3.0e38

_f32 = jnp.float32


def _sc_segment_reduce(A, C, src_s, dt_s, rp):
    mesh = plsc.VectorSubcoreMesh(core_axis_name="c", subcore_axis_name="s")
    out_t = [jax.ShapeDtypeStruct((_NPAD, _H), _f32) for _ in range(4)]
    scratch = [
        pltpu.VMEM((_CHUNK, _H), _f32),
        pltpu.VMEM((_CHUNK, _H), _f32),
        pltpu.VMEM((_CHUNK, _H), _f32),
        pltpu.VMEM((_CHUNK, _H), _f32),
        pltpu.VMEM((8, _H), _f32),
        pltpu.VMEM((_EB,), jnp.int32),
        pltpu.VMEM((_EB + 16,), jnp.int32),
        pltpu.VMEM((_EB, _H), _f32),
        pltpu.VMEM((_NPW + 16,), jnp.int32),
        pltpu.SemaphoreType.DMA,
    ]

    @functools.partial(pl.kernel, mesh=mesh, out_type=out_t,
                       scratch_types=scratch)
    def k(a_h, c_h, src_h, dt_h, rp_h, sum_h, sq_h, mx_h, mn_h,
          acc_s, acc_q, acc_x, acc_n, c_v, idx_v, dt_v, rows_v, rp_v,
          sem):
        wid = lax.axis_index("s") * _NC + lax.axis_index("c")
        base_w = pl.multiple_of(wid * _NPW, _NPW)
        pltpu.sync_copy(rp_h.at[pl.ds(base_w, _NPW + 16)], rp_v)
        pltpu.sync_copy(c_h, c_v)
        for cc in range(_NPW // _CHUNK):
            nbase = base_w + cc * _CHUNK
            e_lo = rp_v[pl.ds(cc * _CHUNK, 16)][0]
            e_hi = rp_v[pl.ds(cc * _CHUNK + _CHUNK, 16)][0]

            def init_body(r, _):
                for j in range(_H // 16):
                    sl = pl.ds(j * 16, 16)
                    acc_s[r, sl] = jnp.zeros((16,), _f32)
                    acc_q[r, sl] = jnp.zeros((16,), _f32)
                    acc_x[r, sl] = jnp.full((16,), _NEG, _f32)
                    acc_n[r, sl] = jnp.full((16,), _POS, _f32)
                return 0

            lax.fori_loop(0, _CHUNK, init_body, 0)

            kb_lo = e_lo >> 7
            kb_hi = (e_hi + (_EB - 1)) >> 7

            def blk_body(kb, _):
                eb0 = pl.multiple_of(kb << 7, _EB)
                pltpu.sync_copy(src_h.at[pl.ds(eb0, _EB)], idx_v)
                pltpu.sync_copy(dt_h.at[pl.ds(eb0, _EB)], dt_v.at[pl.ds(0, _EB)])
                pltpu.async_copy(a_h.at[idx_v], rows_v, sem).wait()
                lo = jnp.maximum(e_lo, eb0) - eb0
                hi = jnp.minimum(e_hi, eb0 + _EB) - eb0

                def e_body(le, _):
                    dt = dt_v[pl.ds(le, 16)][0]
                    lv = (dt >> 2) - nbase
                    t = dt & 3
                    for j in range(_H // 16):
                        sl = pl.ds(j * 16, 16)
                        v = rows_v[le, sl] + c_v[t, sl]
                        plsc.addupdate(acc_s.at[lv, sl], v)
                        plsc.addupdate(acc_q.at[lv, sl], v * v)
                        acc_x[lv, sl] = jnp.maximum(acc_x[lv, sl], v)
                        acc_n[lv, sl] = jnp.minimum(acc_n[lv, sl], v)
                    return 0

                lax.fori_loop(lo, hi, e_body, 0)
                return 0

            lax.fori_loop(kb_lo, kb_hi, blk_body, 0)
            pltpu.sync_copy(acc_s, sum_h.at[pl.ds(nbase, _CHUNK)])
            pltpu.sync_copy(acc_q, sq_h.at[pl.ds(nbase, _CHUNK)])
            pltpu.sync_copy(acc_x, mx_h.at[pl.ds(nbase, _CHUNK)])
            pltpu.sync_copy(acc_n, mn_h.at[pl.ds(nbase, _CHUNK)])

    return k(A, C, src_s, dt_s, rp)


def _dot(a, b):
    return jnp.dot(a, b, preferred_element_type=_f32,
                   precision=lax.Precision.HIGHEST)


def _prep_c_all(bond_pad, wpe_pad):
    def body(bond_ref, wpe_ref, out_ref):
        for l in range(_L):
            out_ref[l] = _dot(bond_ref[...], wpe_ref[l])

    return pl.pallas_call(
        body,
        out_shape=jax.ShapeDtypeStruct((_L, 8, _H), _f32),
    )(bond_pad, wpe_pad)


def _embed_ab(h_idx2, atom_pad, wps, wpd, bp2):
    def body(idx_ref, atom_ref, wps_ref, wpd_ref, bp_ref, h_ref, a_ref, b_ref):
        pid = pl.program_id(0)
        idx = idx_ref[...]
        cols = lax.broadcasted_iota(jnp.int32, (_BN, 128), 1)
        oh = (cols == idx).astype(_f32)
        rows = pid * _BN + lax.broadcasted_iota(jnp.int32, (_BN, 1), 0)
        valid = (rows < _N).astype(_f32)
        h = _dot(oh, atom_ref[...]) * valid
        h_ref[...] = h
        a_ref[...] = _dot(h, wps_ref[...])
        b_ref[...] = _dot(h, wpd_ref[...]) + bp_ref[...]

    grid = _NPAD // _BN
    rowspec = pl.BlockSpec((_BN, _H), lambda i: (i, 0))
    return pl.pallas_call(
        body,
        grid=(grid,),
        in_specs=[
            pl.BlockSpec((_BN, 1), lambda i: (i, 0)),
            pl.BlockSpec((128, _H), lambda i: (0, 0)),
            pl.BlockSpec((_H, _H), lambda i: (0, 0)),
            pl.BlockSpec((_H, _H), lambda i: (0, 0)),
            pl.BlockSpec((1, _H), lambda i: (0, 0)),
        ],
        out_specs=[rowspec, rowspec, rowspec],
        out_shape=[jax.ShapeDtypeStruct((_NPAD, _H), _f32)] * 3,
    )(h_idx2, atom_pad, wps, wpd, bp2)


def _combine(h, B, su, qu, xu, nu, rp0, rp1, snorm, wq, bq, nxt):
    fuse = nxt is not None

    def body(*refs):
        (h_ref, b_ref, s_ref, q_ref, x_ref, n_ref, r0_ref, r1_ref, sn_ref,
         wq_ref, bq_ref) = refs[:11]
        i = 11
        if fuse:
            wps_ref, wpd_ref, bp_ref = refs[i:i + 3]
            i += 3
        hn_ref = refs[i]
        if fuse:
            an_ref, bn_ref = refs[i + 1:i + 3]
        deg = (r1_ref[...] - r0_ref[...]).astype(_f32)
        d = jnp.maximum(deg, 1.0)
        has = deg > 0.0
        B = b_ref[...]
        su = s_ref[...]
        mean = (su + deg * B) / d
        sq = (q_ref[...] + 2.0 * B * su + deg * B * B) / d
        std = jnp.sqrt(jnp.maximum(sq - mean * mean, 0.0) + 1e-5)
        mx = jnp.where(has, x_ref[...] + B, 0.0)
        mn = jnp.where(has, n_ref[...] + B, 0.0)
        logd = jnp.log(d + 1.0)
        amp = logd * (1.0 / _AVG_D_LOG)
        att = _AVG_D_LOG / logd
        agg = jnp.concatenate([mean, mx, mn, std], axis=1)
        h = h_ref[...]
        post = jnp.concatenate([h, agg, agg * amp, agg * att], axis=1)
        ht = _dot(post, wq_ref[...]) + bq_ref[...]
        hn = h + ht * sn_ref[...]
        hn_ref[...] = hn
        if fuse:
            an_ref[...] = _dot(hn, wps_ref[...])
            bn_ref[...] = _dot(hn, wpd_ref[...]) + bp_ref[...]

    grid = _NPAD // _BN
    rowspec = pl.BlockSpec((_BN, _H), lambda i: (i, 0))
    colspec = pl.BlockSpec((_BN, 1), lambda i: (i, 0))
    full = lambda r, c: pl.BlockSpec((r, c), lambda i: (0, 0))
    in_specs = [rowspec] * 6 + [colspec] * 3 + [full(13 * _H, _H), full(1, _H)]
    n_out = 1
    if fuse:
        in_specs += [full(_H, _H), full(_H, _H), full(1, _H)]
        n_out = 3
    args = [h, B, su, qu, xu, nu, rp0, rp1, snorm, wq, bq]
    if fuse:
        args += list(nxt)
    out = pl.pallas_call(
        body,
        grid=(grid,),
        in_specs=in_specs,
        out_specs=[rowspec] * n_out,
        out_shape=[jax.ShapeDtypeStruct((_NPAD, _H), _f32)] * n_out,
    )(*args)
    return out if fuse else (out[0], None, None)


def _readout(h, w1p, b1p, w2p, b2p, w3p, b3p):
    def body(h_ref, w1_ref, b1_ref, w2_ref, b2_ref, w3_ref, b3_ref, out_ref,
             acc_ref):
        i = pl.program_id(0)

        @pl.when(i == 0)
        def _():
            acc_ref[...] = jnp.zeros((8, _H), _f32)

        part = jnp.sum(h_ref[...], axis=0, keepdims=True)
        acc_ref[0:1, :] = acc_ref[0:1, :] + part

        @pl.when(i == pl.num_programs(0) - 1)
        def _():
            hg = acc_ref[0:1, :] * (1.0 / _N)
            x = jnp.maximum(_dot(hg, w1_ref[...]) + b1_ref[...], 0.0)
            x = jnp.maximum(_dot(x, w2_ref[...]) + b2_ref[...], 0.0)
            y = _dot(x, w3_ref[...]) + b3_ref[...]
            out_ref[...] = jnp.broadcast_to(y, (8, _H))

    grid = _NPAD // _BN
    full = lambda r, c: pl.BlockSpec((r, c), lambda i: (0, 0))
    return pl.pallas_call(
        body,
        grid=(grid,),
        in_specs=[pl.BlockSpec((_BN, _H), lambda i: (i, 0))] + [
            full(_H, _H), full(1, _H), full(_H, _H), full(1, _H),
            full(_H, _H), full(1, _H)],
        out_specs=full(8, _H),
        out_shape=jax.ShapeDtypeStruct((8, _H), _f32),
        scratch_shapes=[pltpu.VMEM((8, _H), _f32)],
    )(h, w1p, b1p, w2p, b2p, w3p, b3p)


def _pad_rows(a, rows, fill=0):
    pad = rows - a.shape[0]
    return jnp.pad(a, ((0, pad),) + ((0, 0),) * (a.ndim - 1),
                   constant_values=fill)


def kernel(h, e, edge_index, snorm_n, snorm_e, params):
    del snorm_e
    src = edge_index[0].astype(jnp.int32)
    dst = edge_index[1].astype(jnp.int32)
    et = e.astype(jnp.int32)

    code = dst * 65536 + src * 4 + et
    code_s = jnp.sort(code)
    src_s = (code_s >> 2) & 16383
    dt_s = ((code_s >> 16) << 2) | (code_s & 3)
    row_ptr = jnp.searchsorted(code_s, jnp.arange(_N + 1, dtype=jnp.int32) << 16
                               ).astype(jnp.int32)
    rp_full = _pad_rows(row_ptr, _NPAD + 16, fill=_E)
    rp0 = rp_full[:_NPAD].reshape(_NPAD, 1)
    rp1 = rp_full[1:_NPAD + 1].reshape(_NPAD, 1)

    h_idx2 = _pad_rows(h.astype(jnp.int32).reshape(_N, 1), _NPAD)
    snorm = _pad_rows(snorm_n.astype(_f32), _NPAD)
    atom_pad = _pad_rows(params["atom_emb"], 128)
    bond_pad = jnp.pad(params["bond_emb"], ((0, 4), (0, 112)))
    wps, wpd, wpe, bps = [], [], [], []
    for lp in params["layers"]:
        wps.append(lp["Wp"][:_H])
        wpd.append(lp["Wp"][_H:2 * _H])
        wpe.append(jnp.pad(lp["Wp"][2 * _H:], ((0, 112), (0, 0))))
        bps.append(lp["bp"].reshape(1, _H))
    wpe_all = jnp.stack(wpe)

    c_all = _prep_c_all(bond_pad, wpe_all)
    hcur, A, B = _embed_ab(h_idx2, atom_pad, wps[0], wpd[0], bps[0])

    for l in range(_L):
        lp = params["layers"][l]
        su, qu, xu, nu = _sc_segment_reduce(A, c_all[l], src_s, dt_s, rp_full)
        nxt = ((wps[l + 1], wpd[l + 1], bps[l + 1]) if l + 1 < _L else None)
        hcur, A, B = _combine(hcur, B, su, qu, xu, nu, rp0, rp1, snorm,
                              lp["Wq"], lp["bq"].reshape(1, _H), nxt)

    (w1, b1), (w2, b2), (w3, b3) = params["mlp"]
    w1p = jnp.pad(w1, ((0, 0), (0, 64)))
    b1p = jnp.pad(b1.reshape(1, 64), ((0, 0), (0, 64)))
    w2p = jnp.pad(w2, ((0, 64), (0, 96)))
    b2p = jnp.pad(b2.reshape(1, 32), ((0, 0), (0, 96)))
    w3p = jnp.pad(w3, ((0, 96), (0, 127)))
    b3p = jnp.pad(b3.reshape(1, 1), ((0, 0), (0, 127)))
    y = _readout(hcur, w1p, b1p, w2p, b2p, w3p, b3p)
    return y[0:1, 0:1]
